# baseline (speedup 1.0000x reference)
import jax
import jax.numpy as jnp
from jax.experimental import pallas as pl

B, N, C, K = 16, 20000, 80, 300
R, L = 200, 8000


def _select_body(logits_ref, sig_ref, thresh_ref, ngt_ref):
    x = logits_ref[0]
    s = jax.nn.sigmoid(x)
    sig_ref[0] = s
    key = jax.lax.bitcast_convert_type(s, jnp.int32)

    def count_ge(t):
        return jnp.sum((key >= t).astype(jnp.int32))

    lo0 = jnp.int32(0)
    hi0 = jnp.int32(0x3F800001)

    def step(_, carry):
        lo, hi = carry
        mid = (lo + hi) // 2
        c = count_ge(mid)
        take_hi = c >= K
        return jnp.where(take_hi, mid, lo), jnp.where(take_hi, hi, mid)

    lo, hi = jax.lax.fori_loop(0, 31, step, (lo0, hi0))
    n_gt = count_ge(lo + 1)
    thresh_ref[...] = jnp.broadcast_to(lo, (1, 8, 128))
    ngt_ref[...] = jnp.broadcast_to(n_gt, (1, 8, 128))


def kernel(pred_logits, pred_boxes, orig_target_sizes):
    flat = pred_logits.reshape(B, R, L)
    sig, thresh, n_gt = pl.pallas_call(
        _select_body,
        out_shape=(
            jax.ShapeDtypeStruct((B, R, L), jnp.float32),
            jax.ShapeDtypeStruct((B, 8, 128), jnp.int32),
            jax.ShapeDtypeStruct((B, 8, 128), jnp.int32),
        ),
        grid=(B,),
        in_specs=[pl.BlockSpec((1, R, L), lambda b: (b, 0, 0))],
        out_specs=(
            pl.BlockSpec((1, R, L), lambda b: (b, 0, 0)),
            pl.BlockSpec((1, 8, 128), lambda b: (b, 0, 0)),
            pl.BlockSpec((1, 8, 128), lambda b: (b, 0, 0)),
        ),
    )(flat)
    thresh = thresh[:, 0, :1]
    n_gt = n_gt[:, 0, :1]

    scores = sig.reshape(B, N * C)
    key = jax.lax.bitcast_convert_type(scores, jnp.int32)

    gt = key > thresh
    eq = key == thresh
    need = (K - n_gt).astype(jnp.int32)
    eq_rank = jnp.cumsum(eq.astype(jnp.int32), axis=1)
    mask = gt | (eq & (eq_rank <= need))

    pos = jnp.cumsum(mask.astype(jnp.int32), axis=1) - 1
    pos = jnp.where(mask, pos, K)
    col = jax.lax.broadcasted_iota(jnp.int32, (B, N * C), 1)
    row = jax.lax.broadcasted_iota(jnp.int32, (B, N * C), 0)
    buf = jnp.zeros((B, K + 1), jnp.int32).at[row, pos].set(col, mode="drop")
    cand_idx = buf[:, :K]

    cand_scores = jnp.take_along_axis(scores, cand_idx, axis=1)
    top_scores, order = jax.lax.top_k(cand_scores, K)
    index = jnp.take_along_axis(cand_idx, order, axis=1)

    labels = index % C
    qindex = index // C

    cx = pred_boxes[..., 0]
    cy = pred_boxes[..., 1]
    w = pred_boxes[..., 2]
    h = pred_boxes[..., 3]
    bbox = jnp.stack(
        [cx - 0.5 * w, cy - 0.5 * h, cx + 0.5 * w, cy + 0.5 * h], axis=-1
    )
    scale = jnp.tile(orig_target_sizes, (1, 2))[:, None, :]
    bbox = bbox * scale
    boxes = jnp.take_along_axis(bbox, qindex[..., None], axis=1)
    return (labels, boxes, top_scores)


# speedup vs baseline: 13.1483x; 13.1483x over previous
import jax
import jax.numpy as jnp
from jax.experimental import pallas as pl

B, N, C, K = 16, 20000, 80, 300
R, L = 200, 8000


def _select_body(logits_ref, sig_ref, idx_ref):
    x = logits_ref[0]
    s = jax.nn.sigmoid(x)
    sig_ref[0] = s
    key = jax.lax.bitcast_convert_type(s, jnp.int32)

    def count_ge(t):
        return jnp.sum((key >= t).astype(jnp.int32))

    lo0 = jnp.int32(0)
    hi0 = jnp.int32(0x3F800001)

    def bstep(_, carry):
        lo, hi = carry
        mid = (lo + hi) // 2
        take_hi = count_ge(mid) >= K
        return jnp.where(take_hi, mid, lo), jnp.where(take_hi, hi, mid)

    T, _ = jax.lax.fori_loop(0, 31, bstep, (lo0, hi0))
    n_gt = count_ge(T + 1)
    need = K - n_gt

    eqm = (key == T).astype(jnp.int32)
    fidx = (
        jax.lax.broadcasted_iota(jnp.int32, (R, L), 0) * L
        + jax.lax.broadcasted_iota(jnp.int32, (R, L), 1)
    )

    def istep(_, carry):
        lo, hi = carry
        mid = (lo + hi) // 2
        c = jnp.sum(eqm * (fidx <= mid).astype(jnp.int32))
        ok = c >= need
        return jnp.where(ok, lo, mid), jnp.where(ok, mid, hi)

    _, cut = jax.lax.fori_loop(0, 21, istep, (jnp.int32(-1), jnp.int32(R * L - 1)))

    slot = jax.lax.broadcasted_iota(jnp.int32, (8, 128), 0) * 128 + (
        jax.lax.broadcasted_iota(jnp.int32, (8, 128), 1)
    )
    lane = jax.lax.broadcasted_iota(jnp.int32, (1, L), 1)

    def row_step(r, carry):
        cnt, acc = carry
        rowk = jax.lax.bitcast_convert_type(
            sig_ref[0, pl.ds(r, 1), :], jnp.int32
        )
        rowf = r * L + lane
        m0 = jnp.where(
            (rowk > T) | ((rowk == T) & (rowf <= cut)), jnp.int32(1), jnp.int32(0)
        )

        def cond(c):
            _, m, _ = c
            return jnp.max(m) > 0

        def extract(c):
            n, m, a = c
            p = jnp.argmax(m.astype(jnp.float32)).astype(jnp.int32)
            hit = (lane == p).astype(jnp.int32)
            a = a + jnp.where(slot == n, r * L + p, 0)
            return n + 1, m * (1 - hit), a

        cnt, _, acc = jax.lax.while_loop(cond, extract, (cnt, m0, acc))
        return cnt, acc

    _, acc = jax.lax.fori_loop(
        0, R, row_step, (jnp.int32(0), jnp.zeros((8, 128), jnp.int32))
    )
    idx_ref[0] = acc


def kernel(pred_logits, pred_boxes, orig_target_sizes):
    flat = pred_logits.reshape(B, R, L)
    sig, idx_buf = pl.pallas_call(
        _select_body,
        out_shape=(
            jax.ShapeDtypeStruct((B, R, L), jnp.float32),
            jax.ShapeDtypeStruct((B, 8, 128), jnp.int32),
        ),
        grid=(B,),
        in_specs=[pl.BlockSpec((1, R, L), lambda b: (b, 0, 0))],
        out_specs=(
            pl.BlockSpec((1, R, L), lambda b: (b, 0, 0)),
            pl.BlockSpec((1, 8, 128), lambda b: (b, 0, 0)),
        ),
    )(flat)

    cand_idx = idx_buf.reshape(B, 1024)[:, :K]
    scores = sig.reshape(B, N * C)
    cand_scores = jnp.take_along_axis(scores, cand_idx, axis=1)

    neg, index = jax.lax.sort((-cand_scores, cand_idx), dimension=1, num_keys=2)
    top_scores = -neg

    labels = index % C
    qindex = index // C

    cx = pred_boxes[..., 0]
    cy = pred_boxes[..., 1]
    w = pred_boxes[..., 2]
    h = pred_boxes[..., 3]
    bbox = jnp.stack(
        [cx - 0.5 * w, cy - 0.5 * h, cx + 0.5 * w, cy + 0.5 * h], axis=-1
    )
    scale = jnp.tile(orig_target_sizes, (1, 2))[:, None, :]
    bbox = bbox * scale
    boxes = jnp.take_along_axis(bbox, qindex[..., None], axis=1)
    return (labels, boxes, top_scores)
